# Initial kernel scaffold; baseline (speedup 1.0000x reference)
import jax
import jax.numpy as jnp
from jax.experimental import pallas as pl


def kernel(x, edge_index, edge_attr, batch_vec, W1_rel, b1_rel, W1_root, W2_rel, b2_rel, W2_root, W_lin1, b_lin1, W_lin2, b_lin2):
    raise NotImplementedError("write your pallas kernel here")


# speedup vs baseline: 3.1221x; 3.1221x over previous
import functools

import jax
import jax.numpy as jnp
from jax import lax
from jax.experimental import pallas as pl
from jax.experimental.pallas import tpu as pltpu
from jax.experimental.pallas import tpu_sc as plsc

N = 32768
E = 524288
BS = 1024
E_PER = 32
IN = 128
HID = 256
NC = 10

FC = 32
NUM_TILES = 16
NUM_CORES = 2
TILE_E = E // NUM_TILES
EB = 128
NB = TILE_E // EB
GB = 64
ROWS_PER_TILE = N // NUM_TILES


def _seg_agg(hT, srcm, dstm, wm, zrows, nchunks):
  rounds = nchunks // NUM_CORES
  mesh = plsc.VectorSubcoreMesh(core_axis_name="c", subcore_axis_name="s")

  @functools.partial(
      pl.kernel,
      mesh=mesh,
      compiler_params=pltpu.CompilerParams(use_tc_tiling_on_sc=False),
      out_type=jax.ShapeDtypeStruct((nchunks, N, FC), jnp.float32),
      scratch_types=[
          pltpu.VMEM((GB, EB), jnp.int32),
          pltpu.VMEM((GB, EB), jnp.int32),
          pltpu.VMEM((GB, EB), jnp.float32),
          pltpu.VMEM((EB,), jnp.int32),
          pltpu.VMEM((EB, FC), jnp.float32),
          pltpu.VMEM_SHARED((N, FC), jnp.float32),
          pltpu.SemaphoreType.DMA,
      ],
  )
  def k(hT_hbm, src_hbm, dst_hbm, w_hbm, z_hbm, out_hbm,
        src_v, dst_v, w_v, idx_v, rows_v, acc, sem):
    c = lax.axis_index("c")
    s = lax.axis_index("s")
    for r in range(rounds):
      q = c * rounds + r
      pltpu.sync_copy(z_hbm, acc.at[pl.ds(s * ROWS_PER_TILE, ROWS_PER_TILE)])
      plsc.subcore_barrier()

      def group(og, carry0):
        pltpu.sync_copy(src_hbm.at[s, pl.ds(og * GB, GB)], src_v)
        pltpu.sync_copy(dst_hbm.at[s, pl.ds(og * GB, GB)], dst_v)
        pltpu.sync_copy(w_hbm.at[s, pl.ds(og * GB, GB)], w_v)

        def batch(j, carry):
          def addoff(i, carry2):
            idx_v[pl.ds(i * 16, 16)] = src_v[j, pl.ds(i * 16, 16)] + q * N
            return carry2
          lax.fori_loop(0, EB // 16, addoff, 0, unroll=True)
          pltpu.async_copy(hT_hbm.at[idx_v], rows_v, sem).wait()

          def emul(g, carry2):
            wrow = w_v[j, pl.ds(g * 16, 16)]
            for k in range(16):
              wv = wrow[k]
              e = g * 16 + k
              rows_v[e, pl.ds(0, 16)] = rows_v[e, pl.ds(0, 16)] * wv
              rows_v[e, pl.ds(16, 16)] = rows_v[e, pl.ds(16, 16)] * wv
            return carry2
          lax.fori_loop(0, EB // 16, emul, 0)

          pltpu.sync_copy(rows_v, acc.at[dst_v.at[j]], add=True)
          return carry
        lax.fori_loop(0, GB, batch, 0)
        return carry0
      lax.fori_loop(0, NB // GB, group, 0)
      plsc.subcore_barrier()
      pltpu.sync_copy(
          acc.at[pl.ds(s * ROWS_PER_TILE, ROWS_PER_TILE)],
          out_hbm.at[q, pl.ds(s * ROWS_PER_TILE, ROWS_PER_TILE)])
      plsc.subcore_barrier()

  return k(hT, srcm, dstm, wm, zrows)


def _conv_dense(aggT, h, W_rel, W_root, b, out_chunks):
  Cq = aggT.shape[0]
  Fin = h.shape[1]
  bn = 1024
  grid = (N // bn,)
  want_t = out_chunks > 0

  def body(aggT_ref, h_ref, wrel_ref, wroot_ref, b_ref, o1_ref, *maybe_o2):
    agg = jnp.concatenate([aggT_ref[q] for q in range(Cq)], axis=1)
    acc = lax.dot_general(
        agg, wrel_ref[...], (((1,), (0,)), ((), ())),
        preferred_element_type=jnp.float32)
    acc += lax.dot_general(
        h_ref[...], wroot_ref[...], (((1,), (0,)), ((), ())),
        preferred_element_type=jnp.float32)
    hout = jnp.maximum(acc + b_ref[...], 0.0)
    o1_ref[...] = hout
    if maybe_o2:
      o2_ref = maybe_o2[0]
      for qq in range(out_chunks):
        o2_ref[qq] = hout[:, qq * FC:(qq + 1) * FC]

  out_shape = [jax.ShapeDtypeStruct((N, HID), jnp.float32)]
  out_specs = [pl.BlockSpec((bn, HID), lambda i: (i, 0))]
  if want_t:
    out_shape.append(
        jax.ShapeDtypeStruct((out_chunks, N, FC), jnp.float32))
    out_specs.append(
        pl.BlockSpec((out_chunks, bn, FC), lambda i: (0, i, 0)))

  res = pl.pallas_call(
      body,
      grid=grid,
      in_specs=[
          pl.BlockSpec((Cq, bn, FC), lambda i: (0, i, 0)),
          pl.BlockSpec((bn, Fin), lambda i: (i, 0)),
          pl.BlockSpec((Cq * FC, HID), lambda i: (0, 0)),
          pl.BlockSpec((Fin, HID), lambda i: (0, 0)),
          pl.BlockSpec((1, HID), lambda i: (0, 0)),
      ],
      out_specs=out_specs,
      out_shape=out_shape,
  )(aggT, h, W_rel, W_root, b.reshape(1, HID))
  return res if want_t else res[0]


def _head(h2, W_perm, b1, W2p, b2p):
  K = E_PER * HID
  bb = 128
  flat = h2.reshape(BS, K)

  def body(x_ref, wp_ref, b1_ref, w2_ref, b2_ref, o_ref):
    g = lax.dot_general(
        x_ref[...], wp_ref[...], (((1,), (0,)), ((), ())),
        preferred_element_type=jnp.float32)
    g = jnp.maximum(g + b1_ref[...], 0.0)
    l = lax.dot_general(
        g, w2_ref[...], (((1,), (0,)), ((), ())),
        preferred_element_type=jnp.float32) + b2_ref[...]
    m = jnp.max(l, axis=-1, keepdims=True)
    e = jnp.exp(l - m)
    o_ref[...] = e / jnp.sum(e, axis=-1, keepdims=True)

  return pl.pallas_call(
      body,
      grid=(BS // bb,),
      in_specs=[
          pl.BlockSpec((bb, K), lambda i: (i, 0)),
          pl.BlockSpec((K, HID), lambda i: (0, 0)),
          pl.BlockSpec((1, HID), lambda i: (0, 0)),
          pl.BlockSpec((HID, 128), lambda i: (0, 0)),
          pl.BlockSpec((1, 128), lambda i: (0, 0)),
      ],
      out_specs=pl.BlockSpec((bb, 128), lambda i: (i, 0)),
      out_shape=jax.ShapeDtypeStruct((BS, 128), jnp.float32),
  )(flat, W_perm, b1.reshape(1, HID), W2p, b2p)


def kernel(x, edge_index, edge_attr, batch_vec, W1_rel, b1_rel, W1_root,
           W2_rel, b2_rel, W2_root, W_lin1, b_lin1, W_lin2, b_lin2):
  del batch_vec
  C1 = IN // FC
  C2 = HID // FC

  srcm = edge_index[0].reshape(NUM_TILES, NB, EB)
  dstm = edge_index[1].reshape(NUM_TILES, NB, EB)
  wm = edge_attr.reshape(NUM_TILES, NB, EB)
  zrows = jnp.zeros((ROWS_PER_TILE, FC), jnp.float32)

  xT = x.reshape(N, C1, FC).transpose(1, 0, 2).reshape(C1 * N, FC)
  agg1T = _seg_agg(xT, srcm, dstm, wm, zrows, C1)
  h1, h1T = _conv_dense(agg1T, x, W1_rel, W1_root, b1_rel, C2)
  h1T = h1T.reshape(C2 * N, FC)

  agg2T = _seg_agg(h1T, srcm, dstm, wm, zrows, C2)
  h2 = _conv_dense(agg2T, h1, W2_rel, W2_root, b2_rel, 0)

  W_perm = W_lin1.reshape(HID, E_PER, HID).transpose(1, 0, 2).reshape(
      E_PER * HID, HID)
  W2p = jnp.zeros((HID, 128), jnp.float32).at[:, :NC].set(W_lin2)
  b2p = jnp.full((1, 128), -1e30, jnp.float32).at[0, :NC].set(b_lin2)
  probs = _head(h2, W_perm, b_lin1, W2p, b2p)
  return probs[:, :NC]


# speedup vs baseline: 6.3346x; 2.0290x over previous
import functools

import jax
import jax.numpy as jnp
from jax import lax
from jax.experimental import pallas as pl
from jax.experimental.pallas import tpu as pltpu
from jax.experimental.pallas import tpu_sc as plsc

N = 32768
E = 524288
BS = 1024
E_PER = 32
IN = 128
HID = 256
NC = 10

FC = 32
NUM_TILES = 16
NUM_CORES = 2
TILE_E = E // NUM_TILES
EB = 128
NB = TILE_E // EB
GB = 64
NBUF = 4
ROWS_PER_TILE = N // NUM_TILES


def _seg_agg(hT, srcm, dstm, wm, zrows, nchunks):
  rounds = nchunks // NUM_CORES
  mesh = plsc.VectorSubcoreMesh(core_axis_name="c", subcore_axis_name="s")

  @functools.partial(
      pl.kernel,
      mesh=mesh,
      compiler_params=pltpu.CompilerParams(use_tc_tiling_on_sc=False),
      out_type=jax.ShapeDtypeStruct((nchunks, N, FC), jnp.float32),
      scratch_types=[
          pltpu.VMEM((GB, EB), jnp.int32),
          pltpu.VMEM((GB, EB), jnp.int32),
          pltpu.VMEM((GB, EB), jnp.float32),
          pltpu.VMEM((NBUF, EB), jnp.int32),
          pltpu.VMEM((NBUF, EB, FC), jnp.float32),
          pltpu.VMEM_SHARED((N, FC), jnp.float32),
          [pltpu.SemaphoreType.DMA] * NBUF,
          [pltpu.SemaphoreType.DMA] * NBUF,
      ],
  )
  def k(hT_hbm, src_hbm, dst_hbm, w_hbm, z_hbm, out_hbm,
        src_v, dst_v, w_v, idx_v, rows_v, acc, gsems, ssems):
    c = lax.axis_index("c")
    s = lax.axis_index("s")

    def fill_idx(b, j, q):
      def addoff(i, carry2):
        idx_v[b, pl.ds(i * 16, 16)] = src_v[j, pl.ds(i * 16, 16)] + q * N
        return carry2
      lax.fori_loop(0, EB // 16, addoff, 0, unroll=True)
      pltpu.async_copy(hT_hbm.at[idx_v.at[b]], rows_v.at[b], gsems[b])

    def wait_gather(b):
      pltpu.make_async_copy(
          hT_hbm.at[idx_v.at[b]], rows_v.at[b], gsems[b]).wait()

    def wait_scatter(b, j):
      pltpu.make_async_copy(
          rows_v.at[b], acc.at[dst_v.at[j]], ssems[b]).wait()

    for r in range(rounds):
      q = c * rounds + r
      pltpu.sync_copy(z_hbm, acc.at[pl.ds(s * ROWS_PER_TILE, ROWS_PER_TILE)])
      plsc.subcore_barrier()

      def group(og, carry0):
        pltpu.sync_copy(src_hbm.at[s, pl.ds(og * GB, GB)], src_v)
        pltpu.sync_copy(dst_hbm.at[s, pl.ds(og * GB, GB)], dst_v)
        pltpu.sync_copy(w_hbm.at[s, pl.ds(og * GB, GB)], w_v)
        fill_idx(0, 0, q)
        fill_idx(1, 1, q)

        def quad(jj, carry):
          for kk in range(NBUF):
            j = NBUF * jj + kk
            tb = (kk + 2) % NBUF
            t = j + 2
            if kk < 2:
              @pl.when(jj > 0)
              def _():
                wait_scatter(tb, t - NBUF)
              fill_idx(tb, t, q)
            else:
              @pl.when(jj < GB // NBUF - 1)
              def _():
                wait_scatter(tb, t - NBUF)
                fill_idx(tb, t, q)
            wait_gather(kk)
            def emul(g, carry2):
              wrow = w_v[j, pl.ds(g * 16, 16)]
              for k in range(16):
                wv = wrow[k]
                e = g * 16 + k
                rows_v[kk, e, pl.ds(0, 16)] = rows_v[kk, e, pl.ds(0, 16)] * wv
                rows_v[kk, e, pl.ds(16, 16)] = rows_v[kk, e, pl.ds(16, 16)] * wv
              return carry2
            lax.fori_loop(0, EB // 16, emul, 0)
            pltpu.async_copy(
                rows_v.at[kk], acc.at[dst_v.at[j]], ssems[kk], add=True)
          return carry
        lax.fori_loop(0, GB // NBUF, quad, 0)
        for kk in range(NBUF):
          wait_scatter(kk, GB - NBUF + kk)
        return carry0
      lax.fori_loop(0, NB // GB, group, 0)
      plsc.subcore_barrier()
      pltpu.sync_copy(
          acc.at[pl.ds(s * ROWS_PER_TILE, ROWS_PER_TILE)],
          out_hbm.at[q, pl.ds(s * ROWS_PER_TILE, ROWS_PER_TILE)])
      plsc.subcore_barrier()

  return k(hT, srcm, dstm, wm, zrows)


def _conv_dense(aggT, h, W_rel, W_root, b, out_chunks):
  Cq = aggT.shape[0]
  Fin = h.shape[1]
  bn = 1024
  grid = (N // bn,)
  want_t = out_chunks > 0

  def body(aggT_ref, h_ref, wrel_ref, wroot_ref, b_ref, o1_ref, *maybe_o2):
    agg = jnp.concatenate([aggT_ref[q] for q in range(Cq)], axis=1)
    acc = lax.dot_general(
        agg, wrel_ref[...], (((1,), (0,)), ((), ())),
        preferred_element_type=jnp.float32)
    acc += lax.dot_general(
        h_ref[...], wroot_ref[...], (((1,), (0,)), ((), ())),
        preferred_element_type=jnp.float32)
    hout = jnp.maximum(acc + b_ref[...], 0.0)
    o1_ref[...] = hout
    if maybe_o2:
      o2_ref = maybe_o2[0]
      for qq in range(out_chunks):
        o2_ref[qq] = hout[:, qq * FC:(qq + 1) * FC]

  out_shape = [jax.ShapeDtypeStruct((N, HID), jnp.float32)]
  out_specs = [pl.BlockSpec((bn, HID), lambda i: (i, 0))]
  if want_t:
    out_shape.append(
        jax.ShapeDtypeStruct((out_chunks, N, FC), jnp.float32))
    out_specs.append(
        pl.BlockSpec((out_chunks, bn, FC), lambda i: (0, i, 0)))

  res = pl.pallas_call(
      body,
      grid=grid,
      in_specs=[
          pl.BlockSpec((Cq, bn, FC), lambda i: (0, i, 0)),
          pl.BlockSpec((bn, Fin), lambda i: (i, 0)),
          pl.BlockSpec((Cq * FC, HID), lambda i: (0, 0)),
          pl.BlockSpec((Fin, HID), lambda i: (0, 0)),
          pl.BlockSpec((1, HID), lambda i: (0, 0)),
      ],
      out_specs=out_specs,
      out_shape=out_shape,
  )(aggT, h, W_rel, W_root, b.reshape(1, HID))
  return res if want_t else res[0]


def _head(h2, W_perm, b1, W2p, b2p):
  K = E_PER * HID
  bb = 128
  flat = h2.reshape(BS, K)

  def body(x_ref, wp_ref, b1_ref, w2_ref, b2_ref, o_ref):
    g = lax.dot_general(
        x_ref[...], wp_ref[...], (((1,), (0,)), ((), ())),
        preferred_element_type=jnp.float32)
    g = jnp.maximum(g + b1_ref[...], 0.0)
    l = lax.dot_general(
        g, w2_ref[...], (((1,), (0,)), ((), ())),
        preferred_element_type=jnp.float32) + b2_ref[...]
    m = jnp.max(l, axis=-1, keepdims=True)
    e = jnp.exp(l - m)
    o_ref[...] = e / jnp.sum(e, axis=-1, keepdims=True)

  return pl.pallas_call(
      body,
      grid=(BS // bb,),
      in_specs=[
          pl.BlockSpec((bb, K), lambda i: (i, 0)),
          pl.BlockSpec((K, HID), lambda i: (0, 0)),
          pl.BlockSpec((1, HID), lambda i: (0, 0)),
          pl.BlockSpec((HID, 128), lambda i: (0, 0)),
          pl.BlockSpec((1, 128), lambda i: (0, 0)),
      ],
      out_specs=pl.BlockSpec((bb, 128), lambda i: (i, 0)),
      out_shape=jax.ShapeDtypeStruct((BS, 128), jnp.float32),
  )(flat, W_perm, b1.reshape(1, HID), W2p, b2p)


def kernel(x, edge_index, edge_attr, batch_vec, W1_rel, b1_rel, W1_root,
           W2_rel, b2_rel, W2_root, W_lin1, b_lin1, W_lin2, b_lin2):
  del batch_vec
  C1 = IN // FC
  C2 = HID // FC

  srcm = edge_index[0].reshape(NUM_TILES, NB, EB)
  dstm = edge_index[1].reshape(NUM_TILES, NB, EB)
  wm = edge_attr.reshape(NUM_TILES, NB, EB)
  zrows = jnp.zeros((ROWS_PER_TILE, FC), jnp.float32)

  xT = x.reshape(N, C1, FC).transpose(1, 0, 2).reshape(C1 * N, FC)
  agg1T = _seg_agg(xT, srcm, dstm, wm, zrows, C1)
  h1, h1T = _conv_dense(agg1T, x, W1_rel, W1_root, b1_rel, C2)
  h1T = h1T.reshape(C2 * N, FC)

  agg2T = _seg_agg(h1T, srcm, dstm, wm, zrows, C2)
  h2 = _conv_dense(agg2T, h1, W2_rel, W2_root, b2_rel, 0)

  W_perm = W_lin1.reshape(HID, E_PER, HID).transpose(1, 0, 2).reshape(
      E_PER * HID, HID)
  W2p = jnp.zeros((HID, 128), jnp.float32).at[:, :NC].set(W_lin2)
  b2p = jnp.full((1, 128), -1e30, jnp.float32).at[0, :NC].set(b_lin2)
  probs = _head(h2, W_perm, b_lin1, W2p, b2p)
  return probs[:, :NC]


# speedup vs baseline: 6.5964x; 1.0413x over previous
import functools

import jax
import jax.numpy as jnp
from jax import lax
from jax.experimental import pallas as pl
from jax.experimental.pallas import tpu as pltpu
from jax.experimental.pallas import tpu_sc as plsc

N = 32768
E = 524288
BS = 1024
E_PER = 32
IN = 128
HID = 256
NC = 10

FC = 32
NUM_TILES = 16
NUM_CORES = 2
TILE_E = E // NUM_TILES
EB = 128
NB = TILE_E // EB
GB = 64
NBUF = 4
ROWS_PER_TILE = N // NUM_TILES


def _seg_agg(hT, srcm, dstm, wm, zrows, nchunks):
  rounds = nchunks // NUM_CORES
  mesh = plsc.VectorSubcoreMesh(core_axis_name="c", subcore_axis_name="s")

  @functools.partial(
      pl.kernel,
      mesh=mesh,
      compiler_params=pltpu.CompilerParams(use_tc_tiling_on_sc=False),
      out_type=jax.ShapeDtypeStruct((nchunks, N, FC), jnp.float32),
      scratch_types=[
          pltpu.VMEM((GB, EB), jnp.int32),
          pltpu.VMEM((GB, EB), jnp.int32),
          pltpu.VMEM((GB, EB), jnp.float32),
          pltpu.VMEM((NBUF, EB, FC), jnp.float32),
          pltpu.VMEM_SHARED((N, FC), jnp.float32),
          [pltpu.SemaphoreType.DMA] * NBUF,
          [pltpu.SemaphoreType.DMA] * NBUF,
      ],
  )
  def k(hT_hbm, src_hbm, dst_hbm, w_hbm, z_hbm, out_hbm,
        src_v, dst_v, w_v, rows_v, acc, gsems, ssems):
    c = lax.axis_index("c")
    s = lax.axis_index("s")

    def fire_gather(b, j, q):
      pltpu.async_copy(
          hT_hbm.at[q].at[src_v.at[j]], rows_v.at[b], gsems[b])

    def wait_gather(b, j, q):
      pltpu.make_async_copy(
          hT_hbm.at[q].at[src_v.at[j]], rows_v.at[b], gsems[b]).wait()

    def wait_scatter(b, j):
      pltpu.make_async_copy(
          rows_v.at[b], acc.at[dst_v.at[j]], ssems[b]).wait()

    for r in range(rounds):
      q = c * rounds + r
      pltpu.sync_copy(z_hbm, acc.at[pl.ds(s * ROWS_PER_TILE, ROWS_PER_TILE)])
      plsc.subcore_barrier()

      def group(og, carry0):
        pltpu.sync_copy(src_hbm.at[s, pl.ds(og * GB, GB)], src_v)
        pltpu.sync_copy(dst_hbm.at[s, pl.ds(og * GB, GB)], dst_v)
        pltpu.sync_copy(w_hbm.at[s, pl.ds(og * GB, GB)], w_v)
        fire_gather(0, 0, q)
        fire_gather(1, 1, q)

        def quad(jj, carry):
          for kk in range(NBUF):
            j = NBUF * jj + kk
            tb = (kk + 2) % NBUF
            t = j + 2
            if kk < 2:
              @pl.when(jj > 0)
              def _():
                wait_scatter(tb, t - NBUF)
              fire_gather(tb, t, q)
            else:
              @pl.when(jj < GB // NBUF - 1)
              def _():
                wait_scatter(tb, t - NBUF)
                fire_gather(tb, t, q)
            wait_gather(kk, j, q)
            def emul(g, carry2):
              wrow = w_v[j, pl.ds(g * 16, 16)]
              for k in range(16):
                wv = wrow[k]
                e = g * 16 + k
                rows_v[kk, e, pl.ds(0, 16)] = rows_v[kk, e, pl.ds(0, 16)] * wv
                rows_v[kk, e, pl.ds(16, 16)] = rows_v[kk, e, pl.ds(16, 16)] * wv
              return carry2
            lax.fori_loop(0, EB // 16, emul, 0)
            pltpu.async_copy(
                rows_v.at[kk], acc.at[dst_v.at[j]], ssems[kk], add=True)
          return carry
        lax.fori_loop(0, GB // NBUF, quad, 0)
        for kk in range(NBUF):
          wait_scatter(kk, GB - NBUF + kk)
        return carry0
      lax.fori_loop(0, NB // GB, group, 0)
      plsc.subcore_barrier()
      pltpu.sync_copy(
          acc.at[pl.ds(s * ROWS_PER_TILE, ROWS_PER_TILE)],
          out_hbm.at[q, pl.ds(s * ROWS_PER_TILE, ROWS_PER_TILE)])
      plsc.subcore_barrier()

  return k(hT, srcm, dstm, wm, zrows)


def _conv_dense(aggT, h, W_rel, W_root, b, out_chunks):
  Cq = aggT.shape[0]
  Fin = h.shape[1]
  bn = 1024
  grid = (N // bn,)
  want_t = out_chunks > 0

  def body(aggT_ref, h_ref, wrel_ref, wroot_ref, b_ref, o1_ref, *maybe_o2):
    agg = jnp.concatenate([aggT_ref[q] for q in range(Cq)], axis=1)
    acc = lax.dot_general(
        agg, wrel_ref[...], (((1,), (0,)), ((), ())),
        preferred_element_type=jnp.float32)
    acc += lax.dot_general(
        h_ref[...], wroot_ref[...], (((1,), (0,)), ((), ())),
        preferred_element_type=jnp.float32)
    hout = jnp.maximum(acc + b_ref[...], 0.0)
    o1_ref[...] = hout
    if maybe_o2:
      o2_ref = maybe_o2[0]
      for qq in range(out_chunks):
        o2_ref[qq] = hout[:, qq * FC:(qq + 1) * FC]

  out_shape = [jax.ShapeDtypeStruct((N, HID), jnp.float32)]
  out_specs = [pl.BlockSpec((bn, HID), lambda i: (i, 0))]
  if want_t:
    out_shape.append(
        jax.ShapeDtypeStruct((out_chunks, N, FC), jnp.float32))
    out_specs.append(
        pl.BlockSpec((out_chunks, bn, FC), lambda i: (0, i, 0)))

  res = pl.pallas_call(
      body,
      grid=grid,
      in_specs=[
          pl.BlockSpec((Cq, bn, FC), lambda i: (0, i, 0)),
          pl.BlockSpec((bn, Fin), lambda i: (i, 0)),
          pl.BlockSpec((Cq * FC, HID), lambda i: (0, 0)),
          pl.BlockSpec((Fin, HID), lambda i: (0, 0)),
          pl.BlockSpec((1, HID), lambda i: (0, 0)),
      ],
      out_specs=out_specs,
      out_shape=out_shape,
  )(aggT, h, W_rel, W_root, b.reshape(1, HID))
  return res if want_t else res[0]


def _head(h2, W_lin1, b1, W2p, b2p):
  EG = 8
  h2v = h2.reshape(BS, E_PER, HID)
  w1t = W_lin1.reshape(HID, E_PER, HID).transpose(1, 0, 2)

  def body(x_ref, w1_ref, b1_ref, w2_ref, b2_ref, o_ref, acc_ref):
    eg = pl.program_id(0)
    part = lax.dot_general(
        x_ref[:, 0, :], w1_ref[0], (((1,), (0,)), ((), ())),
        preferred_element_type=jnp.float32)
    for ee in range(1, EG):
      part += lax.dot_general(
          x_ref[:, ee, :], w1_ref[ee], (((1,), (0,)), ((), ())),
          preferred_element_type=jnp.float32)

    @pl.when(eg == 0)
    def _():
      acc_ref[...] = part

    @pl.when(eg > 0)
    def _():
      acc_ref[...] += part

    @pl.when(eg == E_PER // EG - 1)
    def _():
      g = jnp.maximum(acc_ref[...] + b1_ref[...], 0.0)
      l = lax.dot_general(
          g, w2_ref[...], (((1,), (0,)), ((), ())),
          preferred_element_type=jnp.float32) + b2_ref[...]
      m = jnp.max(l, axis=-1, keepdims=True)
      ex = jnp.exp(l - m)
      o_ref[...] = ex / jnp.sum(ex, axis=-1, keepdims=True)

  return pl.pallas_call(
      body,
      grid=(E_PER // EG,),
      in_specs=[
          pl.BlockSpec((BS, EG, HID), lambda e: (0, e, 0)),
          pl.BlockSpec((EG, HID, HID), lambda e: (e, 0, 0)),
          pl.BlockSpec((1, HID), lambda e: (0, 0)),
          pl.BlockSpec((HID, 128), lambda e: (0, 0)),
          pl.BlockSpec((1, 128), lambda e: (0, 0)),
      ],
      out_specs=pl.BlockSpec((BS, 128), lambda e: (0, 0)),
      out_shape=jax.ShapeDtypeStruct((BS, 128), jnp.float32),
      scratch_shapes=[pltpu.VMEM((BS, HID), jnp.float32)],
  )(h2v, w1t, b1.reshape(1, HID), W2p, b2p)


def kernel(x, edge_index, edge_attr, batch_vec, W1_rel, b1_rel, W1_root,
           W2_rel, b2_rel, W2_root, W_lin1, b_lin1, W_lin2, b_lin2):
  del batch_vec
  C1 = IN // FC
  C2 = HID // FC

  srcm = edge_index[0].reshape(NUM_TILES, NB, EB)
  dstm = edge_index[1].reshape(NUM_TILES, NB, EB)
  wm = edge_attr.reshape(NUM_TILES, NB, EB)
  zrows = jnp.zeros((ROWS_PER_TILE, FC), jnp.float32)

  xT = x.reshape(N, C1, FC).transpose(1, 0, 2)
  agg1T = _seg_agg(xT, srcm, dstm, wm, zrows, C1)
  h1, h1T = _conv_dense(agg1T, x, W1_rel, W1_root, b1_rel, C2)

  agg2T = _seg_agg(h1T, srcm, dstm, wm, zrows, C2)
  h2 = _conv_dense(agg2T, h1, W2_rel, W2_root, b2_rel, 0)

  W2p = jnp.zeros((HID, 128), jnp.float32).at[:, :NC].set(W_lin2)
  b2p = jnp.full((1, 128), -1e30, jnp.float32).at[0, :NC].set(b_lin2)
  probs = _head(h2, W_lin1, b_lin1, W2p, b2p)
  return probs[:, :NC]


# speedup vs baseline: 8.8737x; 1.3452x over previous
import functools

import jax
import jax.numpy as jnp
from jax import lax
from jax.experimental import pallas as pl
from jax.experimental.pallas import tpu as pltpu
from jax.experimental.pallas import tpu_sc as plsc

N = 32768
E = 524288
BS = 1024
E_PER = 32
IN = 128
HID = 256
NC = 10

FC = 32
NUM_TILES = 16
NUM_CORES = 2
TILE_E = E // NUM_TILES
EB = 128
NB = TILE_E // EB
GB = 64
NBUF = 4
ROWS_PER_TILE = N // NUM_TILES


def _seg_agg(hs, srcm, dstm, wm, zrows, nchunks):
  na = nchunks * FC // 128
  cpa = 128 // FC
  rounds = nchunks // NUM_CORES
  mesh = plsc.VectorSubcoreMesh(core_axis_name="c", subcore_axis_name="s")

  @functools.partial(
      pl.kernel,
      mesh=mesh,
      compiler_params=pltpu.CompilerParams(use_tc_tiling_on_sc=False),
      out_type=jax.ShapeDtypeStruct((na, N, 128), jnp.float32),
      scratch_types=[
          pltpu.VMEM((GB, EB), jnp.int32),
          pltpu.VMEM((GB, EB), jnp.int32),
          pltpu.VMEM((GB, EB), jnp.float32),
          pltpu.VMEM((NBUF, EB), jnp.int32),
          pltpu.VMEM((NBUF, EB, FC), jnp.float32),
          pltpu.VMEM_SHARED((N, FC), jnp.float32),
          [pltpu.SemaphoreType.DMA] * NBUF,
          [pltpu.SemaphoreType.DMA] * NBUF,
      ],
  )
  def k(h_hbm, src_hbm, dst_hbm, w_hbm, z_hbm, out_hbm,
        src_v, dst_v, w_v, idx_v, rows_v, acc, gsems, ssems):
    c = lax.axis_index("c")
    s = lax.axis_index("s")

    def fire_gather(b, j, aidx, qa):
      def addoff(i, carry2):
        idx_v[b, pl.ds(i * 16, 16)] = src_v[j, pl.ds(i * 16, 16)] * cpa + qa
        return carry2
      lax.fori_loop(0, EB // 16, addoff, 0, unroll=True)
      pltpu.async_copy(
          h_hbm.at[aidx].at[idx_v.at[b]], rows_v.at[b], gsems[b])

    def wait_gather(b, aidx):
      pltpu.make_async_copy(
          h_hbm.at[aidx].at[idx_v.at[b]], rows_v.at[b], gsems[b]).wait()

    def wait_scatter(b, j):
      pltpu.make_async_copy(
          rows_v.at[b], acc.at[dst_v.at[j]], ssems[b]).wait()

    for r in range(rounds):
      q = c * rounds + r
      aidx = (c * na) // NUM_CORES
      qa = q - aidx * cpa
      pltpu.sync_copy(z_hbm, acc.at[pl.ds(s * ROWS_PER_TILE, ROWS_PER_TILE)])
      plsc.subcore_barrier()

      def group(og, carry0):
        pltpu.sync_copy(src_hbm.at[s, pl.ds(og * GB, GB)], src_v)
        pltpu.sync_copy(dst_hbm.at[s, pl.ds(og * GB, GB)], dst_v)
        pltpu.sync_copy(w_hbm.at[s, pl.ds(og * GB, GB)], w_v)
        fire_gather(0, 0, aidx, qa)
        fire_gather(1, 1, aidx, qa)

        def quad(jj, carry):
          for kk in range(NBUF):
            j = NBUF * jj + kk
            tb = (kk + 2) % NBUF
            t = j + 2
            if kk < 2:
              @pl.when(jj > 0)
              def _():
                wait_scatter(tb, t - NBUF)
              fire_gather(tb, t, aidx, qa)
            else:
              @pl.when(jj < GB // NBUF - 1)
              def _():
                wait_scatter(tb, t - NBUF)
                fire_gather(tb, t, aidx, qa)
            wait_gather(kk, aidx)
            def emul(g, carry2):
              wrow = w_v[j, pl.ds(g * 16, 16)]
              for k in range(16):
                wv = wrow[k]
                e = g * 16 + k
                rows_v[kk, e, pl.ds(0, 16)] = rows_v[kk, e, pl.ds(0, 16)] * wv
                rows_v[kk, e, pl.ds(16, 16)] = rows_v[kk, e, pl.ds(16, 16)] * wv
              return carry2
            lax.fori_loop(0, EB // 16, emul, 0)
            pltpu.async_copy(
                rows_v.at[kk], acc.at[dst_v.at[j]], ssems[kk], add=True)
          return carry
        lax.fori_loop(0, GB // NBUF, quad, 0)
        for kk in range(NBUF):
          wait_scatter(kk, GB - NBUF + kk)
        return carry0
      lax.fori_loop(0, NB // GB, group, 0)
      plsc.subcore_barrier()
      pltpu.sync_copy(
          acc.at[pl.ds(s * ROWS_PER_TILE, ROWS_PER_TILE)],
          out_hbm.at[aidx].at[pl.ds(s * ROWS_PER_TILE, ROWS_PER_TILE),
                              pl.ds(qa * FC, FC)])
      plsc.subcore_barrier()

  return k(hs, srcm, dstm, wm, zrows)


def _conv_dense(aggs, hss, W_rel, W_root, b, out_stacked):
  na_a = aggs.shape[0]
  na_h = hss.shape[0]
  bn = 1024

  def body(agg_ref, h_ref, wrel_ref, wroot_ref, b_ref, o_ref):
    agg = jnp.concatenate([agg_ref[a] for a in range(na_a)], axis=1)
    hv = jnp.concatenate([h_ref[a] for a in range(na_h)], axis=1)
    acc = lax.dot_general(
        agg, wrel_ref[...], (((1,), (0,)), ((), ())),
        preferred_element_type=jnp.float32)
    acc += lax.dot_general(
        hv, wroot_ref[...], (((1,), (0,)), ((), ())),
        preferred_element_type=jnp.float32)
    hout = jnp.maximum(acc + b_ref[...], 0.0)
    if out_stacked:
      for a in range(HID // 128):
        o_ref[a] = hout[:, a * 128:(a + 1) * 128]
    else:
      o_ref[...] = hout

  if out_stacked:
    out_shape = jax.ShapeDtypeStruct((HID // 128, N, 128), jnp.float32)
    out_spec = pl.BlockSpec((HID // 128, bn, 128), lambda i: (0, i, 0))
  else:
    out_shape = jax.ShapeDtypeStruct((N, HID), jnp.float32)
    out_spec = pl.BlockSpec((bn, HID), lambda i: (i, 0))

  return pl.pallas_call(
      body,
      grid=(N // bn,),
      in_specs=[
          pl.BlockSpec((na_a, bn, 128), lambda i: (0, i, 0)),
          pl.BlockSpec((na_h, bn, 128), lambda i: (0, i, 0)),
          pl.BlockSpec((na_a * 128, HID), lambda i: (0, 0)),
          pl.BlockSpec((na_h * 128, HID), lambda i: (0, 0)),
          pl.BlockSpec((1, HID), lambda i: (0, 0)),
      ],
      out_specs=out_spec,
      out_shape=out_shape,
  )(aggs, hss, W_rel, W_root, b.reshape(1, HID))


def _head(h2, W_lin1, b1, W2p, b2p):
  EG = 8
  h2v = h2.reshape(BS, E_PER, HID)
  w1t = W_lin1.reshape(HID, E_PER, HID).transpose(1, 0, 2)

  def body(x_ref, w1_ref, b1_ref, w2_ref, b2_ref, o_ref, acc_ref):
    eg = pl.program_id(0)
    part = lax.dot_general(
        x_ref[:, 0, :], w1_ref[0], (((1,), (0,)), ((), ())),
        preferred_element_type=jnp.float32)
    for ee in range(1, EG):
      part += lax.dot_general(
          x_ref[:, ee, :], w1_ref[ee], (((1,), (0,)), ((), ())),
          preferred_element_type=jnp.float32)

    @pl.when(eg == 0)
    def _():
      acc_ref[...] = part

    @pl.when(eg > 0)
    def _():
      acc_ref[...] += part

    @pl.when(eg == E_PER // EG - 1)
    def _():
      g = jnp.maximum(acc_ref[...] + b1_ref[...], 0.0)
      l = lax.dot_general(
          g, w2_ref[...], (((1,), (0,)), ((), ())),
          preferred_element_type=jnp.float32) + b2_ref[...]
      m = jnp.max(l, axis=-1, keepdims=True)
      ex = jnp.exp(l - m)
      o_ref[...] = ex / jnp.sum(ex, axis=-1, keepdims=True)

  return pl.pallas_call(
      body,
      grid=(E_PER // EG,),
      in_specs=[
          pl.BlockSpec((BS, EG, HID), lambda e: (0, e, 0)),
          pl.BlockSpec((EG, HID, HID), lambda e: (e, 0, 0)),
          pl.BlockSpec((1, HID), lambda e: (0, 0)),
          pl.BlockSpec((HID, 128), lambda e: (0, 0)),
          pl.BlockSpec((1, 128), lambda e: (0, 0)),
      ],
      out_specs=pl.BlockSpec((BS, 128), lambda e: (0, 0)),
      out_shape=jax.ShapeDtypeStruct((BS, 128), jnp.float32),
      scratch_shapes=[pltpu.VMEM((BS, HID), jnp.float32)],
  )(h2v, w1t, b1.reshape(1, HID), W2p, b2p)


def kernel(x, edge_index, edge_attr, batch_vec, W1_rel, b1_rel, W1_root,
           W2_rel, b2_rel, W2_root, W_lin1, b_lin1, W_lin2, b_lin2):
  del batch_vec
  C1 = IN // FC
  C2 = HID // FC

  srcm = edge_index[0].reshape(NUM_TILES, NB, EB)
  dstm = edge_index[1].reshape(NUM_TILES, NB, EB)
  wm = edge_attr.reshape(NUM_TILES, NB, EB)
  zrows = jnp.zeros((ROWS_PER_TILE, FC), jnp.float32)

  agg1s = _seg_agg(x.reshape(1, 4 * N, FC), srcm, dstm, wm, zrows, C1)
  h1s = _conv_dense(agg1s, x.reshape(1, N, 128), W1_rel, W1_root,
                    b1_rel, True)

  agg2s = _seg_agg(h1s.reshape(2, 4 * N, FC), srcm, dstm, wm, zrows, C2)
  h2 = _conv_dense(agg2s, h1s, W2_rel, W2_root, b2_rel, False)

  W2p = jnp.zeros((HID, 128), jnp.float32).at[:, :NC].set(W_lin2)
  b2p = jnp.full((1, 128), -1e30, jnp.float32).at[0, :NC].set(b_lin2)
  probs = _head(h2, W_lin1, b_lin1, W2p, b2p)
  return probs[:, :NC]


# speedup vs baseline: 9.2102x; 1.0379x over previous
import functools

import jax
import jax.numpy as jnp
from jax import lax
from jax.experimental import pallas as pl
from jax.experimental.pallas import tpu as pltpu
from jax.experimental.pallas import tpu_sc as plsc

N = 32768
E = 524288
BS = 1024
E_PER = 32
IN = 128
HID = 256
NC = 10

FC = 32
NUM_TILES = 16
NUM_CORES = 2
TILE_E = E // NUM_TILES
EB = 128
NB = TILE_E // EB
GB = 64
NBUF = 4
ROWS_PER_TILE = N // NUM_TILES


def _seg_agg(hs, srcm, dstm, wm, zrows, nchunks):
  na = nchunks * FC // 128
  cpa = 128 // FC
  rounds = nchunks // NUM_CORES
  mesh = plsc.VectorSubcoreMesh(core_axis_name="c", subcore_axis_name="s")

  @functools.partial(
      pl.kernel,
      mesh=mesh,
      compiler_params=pltpu.CompilerParams(use_tc_tiling_on_sc=False),
      out_type=jax.ShapeDtypeStruct((na, N, 128), jnp.float32),
      scratch_types=[
          pltpu.VMEM((GB, EB), jnp.int32),
          pltpu.VMEM((GB, EB), jnp.int32),
          pltpu.VMEM((GB, EB), jnp.float32),
          pltpu.VMEM((NBUF, 2, EB), jnp.int32),
          pltpu.VMEM((NBUF, 2, EB, FC), jnp.float32),
          pltpu.VMEM_SHARED((N, FC), jnp.float32),
          [pltpu.SemaphoreType.DMA] * NBUF,
          [pltpu.SemaphoreType.DMA] * NBUF,
      ],
  )
  def k(h_hbm, src_hbm, dst_hbm, w_hbm, z_hbm, out_hbm,
        src_v, dst_v, w_v, idx_v, rows_v, acc, gsems, ssems):
    c = lax.axis_index("c")
    s = lax.axis_index("s")

    def fire_gather(b, p, aidx, qa):
      for half in range(2):
        j = 2 * p + half

        def addoff(i, carry2):
          idx_v[b, half, pl.ds(i * 16, 16)] = (
              src_v[j, pl.ds(i * 16, 16)] * cpa + qa)
          return carry2
        lax.fori_loop(0, EB // 16, addoff, 0, unroll=True)
        pltpu.async_copy(
            h_hbm.at[aidx].at[idx_v.at[b, half]], rows_v.at[b, half],
            gsems[b])

    def wait_gather(b, aidx):
      for half in range(2):
        pltpu.make_async_copy(
            h_hbm.at[aidx].at[idx_v.at[b, half]], rows_v.at[b, half],
            gsems[b]).wait()

    def fire_scatter(b, p):
      for half in range(2):
        pltpu.async_copy(
            rows_v.at[b, half], acc.at[dst_v.at[2 * p + half]], ssems[b],
            add=True)

    def wait_scatter(b, p):
      for half in range(2):
        pltpu.make_async_copy(
            rows_v.at[b, half], acc.at[dst_v.at[2 * p + half]],
            ssems[b]).wait()

    for r in range(rounds):
      q = c * rounds + r
      aidx = (c * na) // NUM_CORES
      qa = q - aidx * cpa
      pltpu.sync_copy(z_hbm, acc.at[pl.ds(s * ROWS_PER_TILE, ROWS_PER_TILE)])
      plsc.subcore_barrier()

      def group(og, carry0):
        pltpu.sync_copy(src_hbm.at[s, pl.ds(og * GB, GB)], src_v)
        pltpu.sync_copy(dst_hbm.at[s, pl.ds(og * GB, GB)], dst_v)
        pltpu.sync_copy(w_hbm.at[s, pl.ds(og * GB, GB)], w_v)
        fire_gather(0, 0, aidx, qa)
        fire_gather(1, 1, aidx, qa)

        PG = GB // 2

        def quad(jj, carry):
          for kk in range(NBUF):
            p = NBUF * jj + kk
            tb = (kk + 2) % NBUF
            t = p + 2
            if kk < 2:
              @pl.when(jj > 0)
              def _():
                wait_scatter(tb, t - NBUF)
              fire_gather(tb, t, aidx, qa)
            else:
              @pl.when(jj < PG // NBUF - 1)
              def _():
                wait_scatter(tb, t - NBUF)
                fire_gather(tb, t, aidx, qa)
            wait_gather(kk, aidx)
            for half in range(2):
              j = 2 * p + half

              def emul(g, carry2):
                wrow = w_v[j, pl.ds(g * 16, 16)]
                for k in range(16):
                  wv = wrow[k]
                  e = g * 16 + k
                  rows_v[kk, half, e, pl.ds(0, 16)] = (
                      rows_v[kk, half, e, pl.ds(0, 16)] * wv)
                  rows_v[kk, half, e, pl.ds(16, 16)] = (
                      rows_v[kk, half, e, pl.ds(16, 16)] * wv)
                return carry2
              lax.fori_loop(0, EB // 16, emul, 0)
            fire_scatter(kk, p)
          return carry
        lax.fori_loop(0, PG // NBUF, quad, 0)
        for kk in range(NBUF):
          wait_scatter(kk, PG - NBUF + kk)
        return carry0
      lax.fori_loop(0, NB // GB, group, 0)
      plsc.subcore_barrier()
      pltpu.sync_copy(
          acc.at[pl.ds(s * ROWS_PER_TILE, ROWS_PER_TILE)],
          out_hbm.at[aidx].at[pl.ds(s * ROWS_PER_TILE, ROWS_PER_TILE),
                              pl.ds(qa * FC, FC)])
      plsc.subcore_barrier()

  return k(hs, srcm, dstm, wm, zrows)


def _conv_dense(aggs, hss, W_rel, W_root, b, out_stacked):
  na_a = aggs.shape[0]
  na_h = hss.shape[0]
  bn = 1024

  def body(agg_ref, h_ref, wrel_ref, wroot_ref, b_ref, o_ref):
    agg = jnp.concatenate([agg_ref[a] for a in range(na_a)], axis=1)
    hv = jnp.concatenate([h_ref[a] for a in range(na_h)], axis=1)
    acc = lax.dot_general(
        agg, wrel_ref[...], (((1,), (0,)), ((), ())),
        preferred_element_type=jnp.float32)
    acc += lax.dot_general(
        hv, wroot_ref[...], (((1,), (0,)), ((), ())),
        preferred_element_type=jnp.float32)
    hout = jnp.maximum(acc + b_ref[...], 0.0)
    if out_stacked:
      for a in range(HID // 128):
        o_ref[a] = hout[:, a * 128:(a + 1) * 128]
    else:
      o_ref[...] = hout

  if out_stacked:
    out_shape = jax.ShapeDtypeStruct((HID // 128, N, 128), jnp.float32)
    out_spec = pl.BlockSpec((HID // 128, bn, 128), lambda i: (0, i, 0))
  else:
    out_shape = jax.ShapeDtypeStruct((N, HID), jnp.float32)
    out_spec = pl.BlockSpec((bn, HID), lambda i: (i, 0))

  return pl.pallas_call(
      body,
      grid=(N // bn,),
      in_specs=[
          pl.BlockSpec((na_a, bn, 128), lambda i: (0, i, 0)),
          pl.BlockSpec((na_h, bn, 128), lambda i: (0, i, 0)),
          pl.BlockSpec((na_a * 128, HID), lambda i: (0, 0)),
          pl.BlockSpec((na_h * 128, HID), lambda i: (0, 0)),
          pl.BlockSpec((1, HID), lambda i: (0, 0)),
      ],
      out_specs=out_spec,
      out_shape=out_shape,
  )(aggs, hss, W_rel, W_root, b.reshape(1, HID))


def _head(h2, W_lin1, b1, W2p, b2p):
  EG = 8
  h2v = h2.reshape(BS, E_PER, HID)
  w1t = W_lin1.reshape(HID, E_PER, HID).transpose(1, 0, 2)

  def body(x_ref, w1_ref, b1_ref, w2_ref, b2_ref, o_ref, acc_ref):
    eg = pl.program_id(0)
    part = lax.dot_general(
        x_ref[:, 0, :], w1_ref[0], (((1,), (0,)), ((), ())),
        preferred_element_type=jnp.float32)
    for ee in range(1, EG):
      part += lax.dot_general(
          x_ref[:, ee, :], w1_ref[ee], (((1,), (0,)), ((), ())),
          preferred_element_type=jnp.float32)

    @pl.when(eg == 0)
    def _():
      acc_ref[...] = part

    @pl.when(eg > 0)
    def _():
      acc_ref[...] += part

    @pl.when(eg == E_PER // EG - 1)
    def _():
      g = jnp.maximum(acc_ref[...] + b1_ref[...], 0.0)
      l = lax.dot_general(
          g, w2_ref[...], (((1,), (0,)), ((), ())),
          preferred_element_type=jnp.float32) + b2_ref[...]
      m = jnp.max(l, axis=-1, keepdims=True)
      ex = jnp.exp(l - m)
      o_ref[...] = ex / jnp.sum(ex, axis=-1, keepdims=True)

  return pl.pallas_call(
      body,
      grid=(E_PER // EG,),
      in_specs=[
          pl.BlockSpec((BS, EG, HID), lambda e: (0, e, 0)),
          pl.BlockSpec((EG, HID, HID), lambda e: (e, 0, 0)),
          pl.BlockSpec((1, HID), lambda e: (0, 0)),
          pl.BlockSpec((HID, 128), lambda e: (0, 0)),
          pl.BlockSpec((1, 128), lambda e: (0, 0)),
      ],
      out_specs=pl.BlockSpec((BS, 128), lambda e: (0, 0)),
      out_shape=jax.ShapeDtypeStruct((BS, 128), jnp.float32),
      scratch_shapes=[pltpu.VMEM((BS, HID), jnp.float32)],
  )(h2v, w1t, b1.reshape(1, HID), W2p, b2p)


def kernel(x, edge_index, edge_attr, batch_vec, W1_rel, b1_rel, W1_root,
           W2_rel, b2_rel, W2_root, W_lin1, b_lin1, W_lin2, b_lin2):
  del batch_vec
  C1 = IN // FC
  C2 = HID // FC

  srcm = edge_index[0].reshape(NUM_TILES, NB, EB)
  dstm = edge_index[1].reshape(NUM_TILES, NB, EB)
  wm = edge_attr.reshape(NUM_TILES, NB, EB)
  zrows = jnp.zeros((ROWS_PER_TILE, FC), jnp.float32)

  agg1s = _seg_agg(x.reshape(1, 4 * N, FC), srcm, dstm, wm, zrows, C1)
  h1s = _conv_dense(agg1s, x.reshape(1, N, 128), W1_rel, W1_root,
                    b1_rel, True)

  agg2s = _seg_agg(h1s.reshape(2, 4 * N, FC), srcm, dstm, wm, zrows, C2)
  h2 = _conv_dense(agg2s, h1s, W2_rel, W2_root, b2_rel, False)

  W2p = jnp.zeros((HID, 128), jnp.float32).at[:, :NC].set(W_lin2)
  b2p = jnp.full((1, 128), -1e30, jnp.float32).at[0, :NC].set(b_lin2)
  probs = _head(h2, W_lin1, b_lin1, W2p, b2p)
  return probs[:, :NC]


# speedup vs baseline: 9.3147x; 1.0113x over previous
import functools

import jax
import jax.numpy as jnp
from jax import lax
from jax.experimental import pallas as pl
from jax.experimental.pallas import tpu as pltpu
from jax.experimental.pallas import tpu_sc as plsc

N = 32768
E = 524288
BS = 1024
E_PER = 32
IN = 128
HID = 256
NC = 10

FC = 32
NUM_TILES = 16
NUM_CORES = 2
TILE_E = E // NUM_TILES
EB = 128
NB = TILE_E // EB
GB = 64
NBUF = 4
ROWS_PER_TILE = N // NUM_TILES


def _seg_agg(hs, srcm, dstm, wm, zrows, nchunks):
  na = nchunks * FC // 128
  cpa = 128 // FC
  rounds = nchunks // NUM_CORES
  mesh = plsc.VectorSubcoreMesh(core_axis_name="c", subcore_axis_name="s")

  @functools.partial(
      pl.kernel,
      mesh=mesh,
      compiler_params=pltpu.CompilerParams(use_tc_tiling_on_sc=False),
      out_type=jax.ShapeDtypeStruct((na, N, 128), jnp.float32),
      scratch_types=[
          pltpu.VMEM((GB, EB), jnp.int32),
          pltpu.VMEM((GB, EB), jnp.int32),
          pltpu.VMEM((GB, EB), jnp.float32),
          pltpu.VMEM((NBUF, 2, EB), jnp.int32),
          pltpu.VMEM((NBUF, 2, EB, FC), jnp.float32),
          pltpu.VMEM_SHARED((N, FC), jnp.float32),
          [pltpu.SemaphoreType.DMA] * (2 * NBUF),
          [pltpu.SemaphoreType.DMA] * (2 * NBUF),
      ],
  )
  def k(h_hbm, src_hbm, dst_hbm, w_hbm, z_hbm, out_hbm,
        src_v, dst_v, w_v, idx_v, rows_v, acc, gsems, ssems):
    c = lax.axis_index("c")
    s = lax.axis_index("s")

    def fire_gather(b, p, aidx, qa):
      for half in range(2):
        j = 2 * p + half

        def addoff(i, carry2):
          idx_v[b, half, pl.ds(i * 16, 16)] = (
              src_v[j, pl.ds(i * 16, 16)] * cpa + qa)
          return carry2
        lax.fori_loop(0, EB // 16, addoff, 0, unroll=True)
        pltpu.async_copy(
            h_hbm.at[aidx].at[idx_v.at[b, half]], rows_v.at[b, half],
            gsems[2 * b + half])

    def wait_gather_half(b, half, aidx):
      pltpu.make_async_copy(
          h_hbm.at[aidx].at[idx_v.at[b, half]], rows_v.at[b, half],
          gsems[2 * b + half]).wait()

    def fire_scatter_half(b, half, p):
      pltpu.async_copy(
          rows_v.at[b, half], acc.at[dst_v.at[2 * p + half]],
          ssems[2 * b + half], add=True)

    def wait_scatter(b, p):
      for half in range(2):
        pltpu.make_async_copy(
            rows_v.at[b, half], acc.at[dst_v.at[2 * p + half]],
            ssems[2 * b + half]).wait()

    for r in range(rounds):
      q = c * rounds + r
      aidx = (c * na) // NUM_CORES
      qa = q - aidx * cpa
      pltpu.sync_copy(z_hbm, acc.at[pl.ds(s * ROWS_PER_TILE, ROWS_PER_TILE)])
      plsc.subcore_barrier()

      def group(og, carry0):
        pltpu.sync_copy(src_hbm.at[s, pl.ds(og * GB, GB)], src_v)
        pltpu.sync_copy(dst_hbm.at[s, pl.ds(og * GB, GB)], dst_v)
        pltpu.sync_copy(w_hbm.at[s, pl.ds(og * GB, GB)], w_v)
        fire_gather(0, 0, aidx, qa)
        fire_gather(1, 1, aidx, qa)

        PG = GB // 2

        def quad(jj, carry):
          for kk in range(NBUF):
            p = NBUF * jj + kk
            tb = (kk + 2) % NBUF
            t = p + 2
            if kk < 2:
              @pl.when(jj > 0)
              def _():
                wait_scatter(tb, t - NBUF)
              fire_gather(tb, t, aidx, qa)
            else:
              @pl.when(jj < PG // NBUF - 1)
              def _():
                wait_scatter(tb, t - NBUF)
                fire_gather(tb, t, aidx, qa)
            for half in range(2):
              j = 2 * p + half
              wait_gather_half(kk, half, aidx)

              def emul(g, carry2):
                wrow = w_v[j, pl.ds(g * 16, 16)]
                for k in range(16):
                  wv = wrow[k]
                  e = g * 16 + k
                  rows_v[kk, half, e, pl.ds(0, 16)] = (
                      rows_v[kk, half, e, pl.ds(0, 16)] * wv)
                  rows_v[kk, half, e, pl.ds(16, 16)] = (
                      rows_v[kk, half, e, pl.ds(16, 16)] * wv)
                return carry2
              lax.fori_loop(0, EB // 16, emul, 0)
              fire_scatter_half(kk, half, p)
          return carry
        lax.fori_loop(0, PG // NBUF, quad, 0)
        for kk in range(NBUF):
          wait_scatter(kk, PG - NBUF + kk)
        return carry0
      lax.fori_loop(0, NB // GB, group, 0)
      plsc.subcore_barrier()
      pltpu.sync_copy(
          acc.at[pl.ds(s * ROWS_PER_TILE, ROWS_PER_TILE)],
          out_hbm.at[aidx].at[pl.ds(s * ROWS_PER_TILE, ROWS_PER_TILE),
                              pl.ds(qa * FC, FC)])
      plsc.subcore_barrier()

  return k(hs, srcm, dstm, wm, zrows)


def _conv_dense(aggs, hss, W_rel, W_root, b, out_stacked):
  na_a = aggs.shape[0]
  na_h = hss.shape[0]
  bn = 1024

  def body(agg_ref, h_ref, wrel_ref, wroot_ref, b_ref, o_ref):
    agg = jnp.concatenate([agg_ref[a] for a in range(na_a)], axis=1)
    hv = jnp.concatenate([h_ref[a] for a in range(na_h)], axis=1)
    acc = lax.dot_general(
        agg, wrel_ref[...], (((1,), (0,)), ((), ())),
        preferred_element_type=jnp.float32)
    acc += lax.dot_general(
        hv, wroot_ref[...], (((1,), (0,)), ((), ())),
        preferred_element_type=jnp.float32)
    hout = jnp.maximum(acc + b_ref[...], 0.0)
    if out_stacked:
      for a in range(HID // 128):
        o_ref[a] = hout[:, a * 128:(a + 1) * 128]
    else:
      o_ref[...] = hout

  if out_stacked:
    out_shape = jax.ShapeDtypeStruct((HID // 128, N, 128), jnp.float32)
    out_spec = pl.BlockSpec((HID // 128, bn, 128), lambda i: (0, i, 0))
  else:
    out_shape = jax.ShapeDtypeStruct((N, HID), jnp.float32)
    out_spec = pl.BlockSpec((bn, HID), lambda i: (i, 0))

  return pl.pallas_call(
      body,
      grid=(N // bn,),
      in_specs=[
          pl.BlockSpec((na_a, bn, 128), lambda i: (0, i, 0)),
          pl.BlockSpec((na_h, bn, 128), lambda i: (0, i, 0)),
          pl.BlockSpec((na_a * 128, HID), lambda i: (0, 0)),
          pl.BlockSpec((na_h * 128, HID), lambda i: (0, 0)),
          pl.BlockSpec((1, HID), lambda i: (0, 0)),
      ],
      out_specs=out_spec,
      out_shape=out_shape,
  )(aggs, hss, W_rel, W_root, b.reshape(1, HID))


def _head(h2, W_lin1, b1, W2p, b2p):
  EG = 8
  h2v = h2.reshape(BS, E_PER, HID)
  w1t = W_lin1.reshape(HID, E_PER, HID).transpose(1, 0, 2)

  def body(x_ref, w1_ref, b1_ref, w2_ref, b2_ref, o_ref, acc_ref):
    eg = pl.program_id(0)
    part = lax.dot_general(
        x_ref[:, 0, :], w1_ref[0], (((1,), (0,)), ((), ())),
        preferred_element_type=jnp.float32)
    for ee in range(1, EG):
      part += lax.dot_general(
          x_ref[:, ee, :], w1_ref[ee], (((1,), (0,)), ((), ())),
          preferred_element_type=jnp.float32)

    @pl.when(eg == 0)
    def _():
      acc_ref[...] = part

    @pl.when(eg > 0)
    def _():
      acc_ref[...] += part

    @pl.when(eg == E_PER // EG - 1)
    def _():
      g = jnp.maximum(acc_ref[...] + b1_ref[...], 0.0)
      l = lax.dot_general(
          g, w2_ref[...], (((1,), (0,)), ((), ())),
          preferred_element_type=jnp.float32) + b2_ref[...]
      m = jnp.max(l, axis=-1, keepdims=True)
      ex = jnp.exp(l - m)
      o_ref[...] = ex / jnp.sum(ex, axis=-1, keepdims=True)

  return pl.pallas_call(
      body,
      grid=(E_PER // EG,),
      in_specs=[
          pl.BlockSpec((BS, EG, HID), lambda e: (0, e, 0)),
          pl.BlockSpec((EG, HID, HID), lambda e: (e, 0, 0)),
          pl.BlockSpec((1, HID), lambda e: (0, 0)),
          pl.BlockSpec((HID, 128), lambda e: (0, 0)),
          pl.BlockSpec((1, 128), lambda e: (0, 0)),
      ],
      out_specs=pl.BlockSpec((BS, 128), lambda e: (0, 0)),
      out_shape=jax.ShapeDtypeStruct((BS, 128), jnp.float32),
      scratch_shapes=[pltpu.VMEM((BS, HID), jnp.float32)],
  )(h2v, w1t, b1.reshape(1, HID), W2p, b2p)


def kernel(x, edge_index, edge_attr, batch_vec, W1_rel, b1_rel, W1_root,
           W2_rel, b2_rel, W2_root, W_lin1, b_lin1, W_lin2, b_lin2):
  del batch_vec
  C1 = IN // FC
  C2 = HID // FC

  srcm = edge_index[0].reshape(NUM_TILES, NB, EB)
  dstm = edge_index[1].reshape(NUM_TILES, NB, EB)
  wm = edge_attr.reshape(NUM_TILES, NB, EB)
  zrows = jnp.zeros((ROWS_PER_TILE, FC), jnp.float32)

  agg1s = _seg_agg(x.reshape(1, 4 * N, FC), srcm, dstm, wm, zrows, C1)
  h1s = _conv_dense(agg1s, x.reshape(1, N, 128), W1_rel, W1_root,
                    b1_rel, True)

  agg2s = _seg_agg(h1s.reshape(2, 4 * N, FC), srcm, dstm, wm, zrows, C2)
  h2 = _conv_dense(agg2s, h1s, W2_rel, W2_root, b2_rel, False)

  W2p = jnp.zeros((HID, 128), jnp.float32).at[:, :NC].set(W_lin2)
  b2p = jnp.full((1, 128), -1e30, jnp.float32).at[0, :NC].set(b_lin2)
  probs = _head(h2, W_lin1, b_lin1, W2p, b2p)
  return probs[:, :NC]
